# baseline (speedup 1.0000x reference)
import jax
import jax.numpy as jnp
from jax.experimental import pallas as pl


def _ln(x, g, b, eps=1e-5):
    m = jnp.mean(x, axis=-1, keepdims=True)
    v = jnp.mean((x - m) ** 2, axis=-1, keepdims=True)
    return (x - m) / jnp.sqrt(v + eps) * g + b


def _score_keys(x, g, b, w1, b1, w2, b2):
    s = _ln(x, g, b)
    s = jax.nn.gelu(jnp.dot(s, w1) + b1)
    logits = jnp.dot(s, w2) + b2
    m = jnp.max(logits, axis=-1, keepdims=True)
    e = jnp.exp(logits - m)
    score = e[:, 0:1] / jnp.sum(e, axis=-1, keepdims=True)
    return jax.lax.bitcast_convert_type(score, jnp.int32)


def _thresh_kernel(x_ref, g_ref, b_ref, w1_ref, b1_ref, w2_ref, b2_ref,
                   v_ref, t_ref, *, num_keep):
    R, N, C = x_ref.shape
    keys = _score_keys(x_ref[...].reshape(R * N, C), g_ref[...], b_ref[...],
                       w1_ref[...], b1_ref[...], w2_ref[...], b2_ref[...])
    keys = keys.reshape(R, N, 1)
    k = jnp.int32(num_keep)

    def body_val(_, c):
        lo, hi = c
        mid = (lo + hi) // 2
        cnt = jnp.sum((keys >= mid).astype(jnp.int32), axis=(1, 2),
                      keepdims=True)
        ge = cnt >= k
        return jnp.where(ge, mid, lo), jnp.where(ge, hi, mid)

    lo0 = jnp.zeros((R, 1, 1), jnp.int32)
    hi0 = jnp.full((R, 1, 1), 0x3F800001, jnp.int32)
    v, _ = jax.lax.fori_loop(0, 31, body_val, (lo0, hi0))

    n_gt = jnp.sum((keys > v).astype(jnp.int32), axis=(1, 2), keepdims=True)
    r = k - n_gt
    tie = keys == v
    idx = jax.lax.broadcasted_iota(jnp.int32, (R, N, 1), 1)

    def body_idx(_, c):
        lo, hi = c
        mid = (lo + hi) // 2
        cnt = jnp.sum((tie & (idx < mid)).astype(jnp.int32), axis=(1, 2),
                      keepdims=True)
        ok = cnt >= r
        return jnp.where(ok, lo, mid + 1), jnp.where(ok, mid, hi)

    t_idx, _ = jax.lax.fori_loop(
        0, 11, body_idx, (jnp.zeros((R, 1, 1), jnp.int32),
                          jnp.full((R, 1, 1), N, jnp.int32)))

    v_ref[...] = v[:, :, 0]
    t_ref[...] = t_idx[:, :, 0]


def _mlp_kernel(x_ref, v_ref, t_ref, pg_ref, pb_ref, pw1_ref, pb1_ref,
                pw2_ref, pb2_ref, ln2_g, ln2_b, mlp_w1, mlp_b1, mlp_w2,
                mlp_b2, fast_ln_g, fast_ln_b, fast_w1, fast_b1, fast_w2,
                fast_b2, out_ref):
    R, N, C = x_ref.shape
    x = x_ref[...].reshape(R * N, C)

    keys = _score_keys(x, pg_ref[...], pb_ref[...], pw1_ref[...],
                       pb1_ref[...], pw2_ref[...], pb2_ref[...])
    v = jnp.broadcast_to(v_ref[...][:, None, :], (R, N, 1)).reshape(R * N, 1)
    t = jnp.broadcast_to(t_ref[...][:, None, :], (R, N, 1)).reshape(R * N, 1)
    idx = jax.lax.broadcasted_iota(jnp.int32, (R, N, 1), 1).reshape(R * N, 1)
    keep = (keys > v) | ((keys == v) & (idx < t))

    h = _ln(x, ln2_g[...], ln2_b[...])
    h = jnp.dot(jax.nn.gelu(jnp.dot(h, mlp_w1[...]) + mlp_b1[...]),
                mlp_w2[...]) + mlp_b2[...]
    h2 = _ln(x, fast_ln_g[...], fast_ln_b[...])
    h2 = jnp.dot(jax.nn.gelu(jnp.dot(h2, fast_w1[...]) + fast_b1[...]),
                 fast_w2[...]) + fast_b2[...]
    out_ref[...] = (x + jnp.where(keep, h, h2)).reshape(R, N, C)


def _full(a):
    return pl.BlockSpec(a.shape, lambda i: (0,) * a.ndim)


def kernel(x, pred_ln_g, pred_ln_b, pred_w1, pred_b1, pred_w2, pred_b2,
           ln2_g, ln2_b, mlp_w1, mlp_b1, mlp_w2, mlp_b2,
           fast_ln_g, fast_ln_b, fast_w1, fast_b1, fast_w2, fast_b2):
    B, N, C = x.shape
    num_keep = N // 2
    R = 8
    import functools

    r2 = lambda a: a.reshape(1, -1)
    pred_args = (r2(pred_ln_g), r2(pred_ln_b), pred_w1, r2(pred_b1),
                 pred_w2, r2(pred_b2))
    pred_specs = [_full(a) for a in pred_args]

    v_arr, t_arr = pl.pallas_call(
        functools.partial(_thresh_kernel, num_keep=num_keep),
        grid=(B // R,),
        in_specs=[pl.BlockSpec((R, N, C), lambda i: (i, 0, 0))] + pred_specs,
        out_specs=[pl.BlockSpec((R, 1), lambda i: (i, 0)),
                   pl.BlockSpec((R, 1), lambda i: (i, 0))],
        out_shape=[jax.ShapeDtypeStruct((B, 1), jnp.int32),
                   jax.ShapeDtypeStruct((B, 1), jnp.int32)],
    )(x, *pred_args)

    mlp_args = (r2(ln2_g), r2(ln2_b), mlp_w1, r2(mlp_b1), mlp_w2,
                r2(mlp_b2), r2(fast_ln_g), r2(fast_ln_b), fast_w1,
                r2(fast_b1), fast_w2, r2(fast_b2))
    out = pl.pallas_call(
        _mlp_kernel,
        grid=(B // R,),
        in_specs=([pl.BlockSpec((R, N, C), lambda i: (i, 0, 0)),
                   pl.BlockSpec((R, 1), lambda i: (i, 0)),
                   pl.BlockSpec((R, 1), lambda i: (i, 0))]
                  + pred_specs + [_full(a) for a in mlp_args]),
        out_specs=pl.BlockSpec((R, N, C), lambda i: (i, 0, 0)),
        out_shape=jax.ShapeDtypeStruct((B, N, C), x.dtype),
    )(x, v_arr, t_arr, *pred_args, *mlp_args)

    return out


# speedup vs baseline: 2.2378x; 2.2378x over previous
import functools

import jax
import jax.numpy as jnp
from jax.experimental import pallas as pl


def _ln(x, g, b, eps=1e-5):
    m = jnp.mean(x, axis=-1, keepdims=True)
    v = jnp.mean((x - m) ** 2, axis=-1, keepdims=True)
    return (x - m) / jnp.sqrt(v + eps) * g + b


def _col_to_rows(col, r, l):
    return jnp.transpose(col.reshape(r, l, 1), (0, 2, 1)).reshape(r, l)


def _rows_to_col(rows):
    r, l = rows.shape
    return jnp.transpose(rows.reshape(r, 1, l), (0, 2, 1)).reshape(r * l, 1)


def _score_kernel(x_ref, g_ref, b_ref, w1_ref, b1_ref, w2_ref, b2_ref,
                  score_ref):
    T = x_ref.shape[0]
    R, L = score_ref.shape
    s = _ln(x_ref[...], g_ref[...], b_ref[...])
    s = jax.nn.gelu(jnp.dot(s, w1_ref[...]) + b1_ref[...])
    logits = jnp.dot(s, w2_ref[...]) + b2_ref[...]
    m = jnp.max(logits, axis=-1, keepdims=True)
    e = jnp.exp(logits - m)
    score = e[:, 0:1] / jnp.sum(e, axis=-1, keepdims=True)
    score_ref[...] = _col_to_rows(score, R, L)


def _mask_kernel(score_ref, mask_ref, *, num_keep):
    B, N = score_ref.shape
    keys = jax.lax.bitcast_convert_type(score_ref[...], jnp.int32)
    k = jnp.int32(num_keep)

    def body_val(_, c):
        lo, hi = c
        mid = (lo + hi) // 2
        ge = jnp.sum((keys >= mid).astype(jnp.int32), axis=1,
                     keepdims=True) >= k
        return jnp.where(ge, mid, lo), jnp.where(ge, hi, mid)

    lo0 = jnp.zeros((B, 1), jnp.int32)
    hi0 = jnp.full((B, 1), 0x3F800001, jnp.int32)
    v, _ = jax.lax.fori_loop(0, 31, body_val, (lo0, hi0))

    n_gt = jnp.sum((keys > v).astype(jnp.int32), axis=1, keepdims=True)
    r = k - n_gt
    tie = keys == v
    idx = jax.lax.broadcasted_iota(jnp.int32, (B, N), 1)

    def body_idx(_, c):
        lo, hi = c
        mid = (lo + hi) // 2
        cnt = jnp.sum((tie & (idx < mid)).astype(jnp.int32), axis=1,
                      keepdims=True)
        ok = cnt >= r
        return jnp.where(ok, lo, mid + 1), jnp.where(ok, mid, hi)

    t_idx, _ = jax.lax.fori_loop(
        0, 11, body_idx, (jnp.zeros((B, 1), jnp.int32),
                          jnp.full((B, 1), N, jnp.int32)))

    keep = (keys > v) | (tie & (idx < t_idx))
    mask_ref[...] = keep.astype(jnp.float32)


def _mlp_kernel(x_ref, mask_ref, ln2_g, ln2_b, mlp_w1, mlp_b1, mlp_w2,
                mlp_b2, fast_ln_g, fast_ln_b, fast_w1, fast_b1, fast_w2,
                fast_b2, out_ref):
    x = x_ref[...]
    keep = _rows_to_col(mask_ref[...]) > 0.5
    h = _ln(x, ln2_g[...], ln2_b[...])
    h = jnp.dot(jax.nn.gelu(jnp.dot(h, mlp_w1[...]) + mlp_b1[...]),
                mlp_w2[...]) + mlp_b2[...]
    h2 = _ln(x, fast_ln_g[...], fast_ln_b[...])
    h2 = jnp.dot(jax.nn.gelu(jnp.dot(h2, fast_w1[...]) + fast_b1[...]),
                 fast_w2[...]) + fast_b2[...]
    out_ref[...] = x + jnp.where(keep, h, h2)


def _full(a):
    return pl.BlockSpec(a.shape, lambda i: (0,) * a.ndim)


def kernel(x, pred_ln_g, pred_ln_b, pred_w1, pred_b1, pred_w2, pred_b2,
           ln2_g, ln2_b, mlp_w1, mlp_b1, mlp_w2, mlp_b2,
           fast_ln_g, fast_ln_b, fast_w1, fast_b1, fast_w2, fast_b2):
    B, N, C = x.shape
    num_keep = N // 2
    M = B * N
    R = 8
    T = R * N
    xf = x.reshape(M, C)

    r2 = lambda a: a.reshape(1, -1)

    pred_args = (r2(pred_ln_g), r2(pred_ln_b), pred_w1, r2(pred_b1),
                 pred_w2, r2(pred_b2))
    scores = pl.pallas_call(
        _score_kernel,
        grid=(B // R,),
        in_specs=[pl.BlockSpec((T, C), lambda i: (i, 0))]
                 + [_full(a) for a in pred_args],
        out_specs=pl.BlockSpec((R, N), lambda i: (i, 0)),
        out_shape=jax.ShapeDtypeStruct((B, N), jnp.float32),
    )(xf, *pred_args)

    mask = pl.pallas_call(
        functools.partial(_mask_kernel, num_keep=num_keep),
        in_specs=[pl.BlockSpec((B, N), lambda: (0, 0))],
        out_specs=pl.BlockSpec((B, N), lambda: (0, 0)),
        out_shape=jax.ShapeDtypeStruct((B, N), jnp.float32),
    )(scores)

    mlp_args = (r2(ln2_g), r2(ln2_b), mlp_w1, r2(mlp_b1), mlp_w2,
                r2(mlp_b2), r2(fast_ln_g), r2(fast_ln_b), fast_w1,
                r2(fast_b1), fast_w2, r2(fast_b2))
    out = pl.pallas_call(
        _mlp_kernel,
        grid=(B // R,),
        in_specs=([pl.BlockSpec((T, C), lambda i: (i, 0)),
                   pl.BlockSpec((R, N), lambda i: (i, 0))]
                  + [_full(a) for a in mlp_args]),
        out_specs=pl.BlockSpec((T, C), lambda i: (i, 0)),
        out_shape=jax.ShapeDtypeStruct((M, C), x.dtype),
    )(xf, mask, *mlp_args)

    return out.reshape(B, N, C)


# speedup vs baseline: 2.2524x; 1.0065x over previous
import functools

import jax
import jax.numpy as jnp
from jax.experimental import pallas as pl


def _ln(x, g, b, eps=1e-5):
    m = jnp.mean(x, axis=-1, keepdims=True)
    v = jnp.mean((x - m) ** 2, axis=-1, keepdims=True)
    return (x - m) / jnp.sqrt(v + eps) * g + b


def _col_to_rows(col, r, l):
    return jnp.transpose(col.reshape(r, l, 1), (0, 2, 1)).reshape(r, l)


def _rows_to_col(rows):
    r, l = rows.shape
    return jnp.transpose(rows.reshape(r, 1, l), (0, 2, 1)).reshape(r * l, 1)


def _score_kernel(x_ref, g_ref, b_ref, w1_ref, b1_ref, w2_ref, b2_ref,
                  score_ref):
    R, N, C = x_ref.shape
    _, L = score_ref.shape
    s = _ln(x_ref[...].reshape(R * N, C), g_ref[...], b_ref[...])
    s = jax.nn.gelu(jnp.dot(s, w1_ref[...]) + b1_ref[...])
    logits = jnp.dot(s, w2_ref[...]) + b2_ref[...]
    m = jnp.max(logits, axis=-1, keepdims=True)
    e = jnp.exp(logits - m)
    score = e[:, 0:1] / jnp.sum(e, axis=-1, keepdims=True)
    score_ref[...] = _col_to_rows(score, R, L)


def _mask_kernel(score_ref, mask_ref, *, num_keep):
    B, N = score_ref.shape
    keys = jax.lax.bitcast_convert_type(score_ref[...], jnp.int32)
    k = jnp.int32(num_keep)

    def body_val(_, c):
        lo, hi = c
        mid = (lo + hi) // 2
        ge = jnp.sum((keys >= mid).astype(jnp.int32), axis=1,
                     keepdims=True) >= k
        return jnp.where(ge, mid, lo), jnp.where(ge, hi, mid)

    lo0 = jnp.zeros((B, 1), jnp.int32)
    hi0 = jnp.full((B, 1), 0x3F800001, jnp.int32)
    v, _ = jax.lax.fori_loop(0, 31, body_val, (lo0, hi0))

    n_gt = jnp.sum((keys > v).astype(jnp.int32), axis=1, keepdims=True)
    r = k - n_gt
    tie = keys == v
    idx = jax.lax.broadcasted_iota(jnp.int32, (B, N), 1)

    def body_idx(_, c):
        lo, hi = c
        mid = (lo + hi) // 2
        cnt = jnp.sum((tie & (idx < mid)).astype(jnp.int32), axis=1,
                      keepdims=True)
        ok = cnt >= r
        return jnp.where(ok, lo, mid + 1), jnp.where(ok, mid, hi)

    t_idx, _ = jax.lax.fori_loop(
        0, 11, body_idx, (jnp.zeros((B, 1), jnp.int32),
                          jnp.full((B, 1), N, jnp.int32)))

    keep = (keys > v) | (tie & (idx < t_idx))
    mask_ref[...] = keep.astype(jnp.float32)


def _mlp_kernel(x_ref, mask_ref, ln2_g, ln2_b, mlp_w1, mlp_b1, mlp_w2,
                mlp_b2, fast_ln_g, fast_ln_b, fast_w1, fast_b1, fast_w2,
                fast_b2, out_ref):
    R, N, C = x_ref.shape
    x = x_ref[...].reshape(R * N, C)
    keep = _rows_to_col(mask_ref[...]) > 0.5
    h = _ln(x, ln2_g[...], ln2_b[...])
    h = jnp.dot(jax.nn.gelu(jnp.dot(h, mlp_w1[...]) + mlp_b1[...]),
                mlp_w2[...]) + mlp_b2[...]
    h2 = _ln(x, fast_ln_g[...], fast_ln_b[...])
    h2 = jnp.dot(jax.nn.gelu(jnp.dot(h2, fast_w1[...]) + fast_b1[...]),
                 fast_w2[...]) + fast_b2[...]
    out_ref[...] = (x + jnp.where(keep, h, h2)).reshape(R, N, C)


def _full(a):
    return pl.BlockSpec(a.shape, lambda i: (0,) * a.ndim)


def kernel(x, pred_ln_g, pred_ln_b, pred_w1, pred_b1, pred_w2, pred_b2,
           ln2_g, ln2_b, mlp_w1, mlp_b1, mlp_w2, mlp_b2,
           fast_ln_g, fast_ln_b, fast_w1, fast_b1, fast_w2, fast_b2):
    B, N, C = x.shape
    num_keep = N // 2
    R = 8

    r2 = lambda a: a.reshape(1, -1)

    pred_args = (r2(pred_ln_g), r2(pred_ln_b), pred_w1, r2(pred_b1),
                 pred_w2, r2(pred_b2))
    scores = pl.pallas_call(
        _score_kernel,
        grid=(B // R,),
        in_specs=[pl.BlockSpec((R, N, C), lambda i: (i, 0, 0))]
                 + [_full(a) for a in pred_args],
        out_specs=pl.BlockSpec((R, N), lambda i: (i, 0)),
        out_shape=jax.ShapeDtypeStruct((B, N), jnp.float32),
    )(x, *pred_args)

    mask = pl.pallas_call(
        functools.partial(_mask_kernel, num_keep=num_keep),
        in_specs=[pl.BlockSpec((B, N), lambda: (0, 0))],
        out_specs=pl.BlockSpec((B, N), lambda: (0, 0)),
        out_shape=jax.ShapeDtypeStruct((B, N), jnp.float32),
    )(scores)

    mlp_args = (r2(ln2_g), r2(ln2_b), mlp_w1, r2(mlp_b1), mlp_w2,
                r2(mlp_b2), r2(fast_ln_g), r2(fast_ln_b), fast_w1,
                r2(fast_b1), fast_w2, r2(fast_b2))
    out = pl.pallas_call(
        _mlp_kernel,
        grid=(B // R,),
        in_specs=([pl.BlockSpec((R, N, C), lambda i: (i, 0, 0)),
                   pl.BlockSpec((R, N), lambda i: (i, 0))]
                  + [_full(a) for a in mlp_args]),
        out_specs=pl.BlockSpec((R, N, C), lambda i: (i, 0, 0)),
        out_shape=jax.ShapeDtypeStruct((B, N, C), x.dtype),
    )(x, mask, *mlp_args)

    return out
